# baseline (speedup 1.0000x reference)
import functools

import jax
import jax.numpy as jnp
from jax import lax
from jax.experimental import pallas as pl
from jax.experimental.pallas import tpu as pltpu
from jax.experimental.pallas import tpu_sc as plsc

N = 10000
D = 128
E = 320000
EPSILON = 0.1

NC = 2
NS = 16
NW = NC * NS
EPB = 64
NB = 160
RING = 4
EPW = EPB * NB
E_PAD = EPW * NW
ROWS_SH = 10240
RPT = ROWS_SH // NS

_sc_mesh = plsc.VectorSubcoreMesh(core_axis_name="c", subcore_axis_name="s")


@functools.partial(
    pl.kernel,
    mesh=_sc_mesh,
    compiler_params=pltpu.CompilerParams(
        use_tc_tiling_on_sc=False, needs_layout_passes=False),
    out_type=jax.ShapeDtypeStruct((NC, ROWS_SH, D), jnp.float32),
    scratch_types=[
        pltpu.VMEM((EPW,), jnp.int32),
        pltpu.VMEM((4, EPB), jnp.int32),
        pltpu.VMEM((4, EPB), jnp.int32),
        pltpu.VMEM((3, EPB, D // 2), jnp.int32),
        pltpu.VMEM((2, EPB, D), jnp.float32),
        pltpu.VMEM_SHARED((ROWS_SH, D), jnp.float32),
        pltpu.SemaphoreType.DMA,
        pltpu.SemaphoreType.DMA((3,)),
        pltpu.SemaphoreType.DMA((2,)),
    ],
)
def _sc_agg(pk_hbm, feat_hbm, out_hbm,
            pk_v, srcr, dstr, bufb, buff, acc, sem0, sems, sem2):
    cid = lax.axis_index("c")
    sid = lax.axis_index("s")
    wid = sid * NC + cid

    pltpu.sync_copy(pk_hbm.at[wid], pk_v)

    zero16 = jnp.zeros((16,), jnp.float32)

    def _zrow(i, carry):
        for k in range(D // 16):
            buff[0, i, pl.ds(k * 16, 16)] = zero16
        return carry

    lax.fori_loop(0, EPB, _zrow, 0)
    for b in range(RPT // EPB):
        pltpu.sync_copy(buff.at[0], acc.at[pl.ds(sid * RPT + b * EPB, EPB)])
    plsc.subcore_barrier()

    msk_hi = jnp.full((16,), -65536, dtype=jnp.int32)

    def _body(j, carry):
        pari = lax.rem(j, 4)
        parf = lax.rem(j, 2)
        parb = lax.rem(j, 3)

        @pl.when(j >= 4)
        def _wscat():
            pltpu.make_async_copy(
                buff.at[parf], acc.at[dstr.at[pari]], sem2.at[parf]).wait()

        @pl.when(jnp.logical_and(j >= 2, j < NB + 2))
        def _scat():
            parb2 = lax.rem(j + 1, 3)
            pari2 = lax.rem(j + 2, 4)
            pltpu.make_async_copy(
                feat_hbm.at[srcr.at[pari2]], bufb.at[parb2],
                sems.at[parb2]).wait()
            for r in range(EPB):
                for g in range(D // 32):
                    v = bufb[parb2, r, pl.ds(16 * g, 16)]
                    lo = plsc.bitcast(v << 16, jnp.float32)
                    hi = plsc.bitcast(v & msk_hi, jnp.float32)
                    buff[parf, r, pl.ds(32 * g, 16)] = lo
                    buff[parf, r, pl.ds(32 * g + 16, 16)] = hi
            pltpu.async_copy(
                buff.at[parf], acc.at[dstr.at[pari2]], sem2.at[parf],
                add=True)

        @pl.when(j < NB)
        def _issue():
            for k in range(EPB // 16):
                p = pk_v[pl.ds(j * EPB + k * 16, 16)]
                srcr[pari, pl.ds(k * 16, 16)] = p & 0xFFFF
                dstr[pari, pl.ds(k * 16, 16)] = p >> 16
            pltpu.async_copy(
                feat_hbm.at[srcr.at[pari]], bufb.at[parb], sems.at[parb])

        return carry

    lax.fori_loop(0, NB + 4, _body, 0)
    plsc.subcore_barrier()

    pltpu.sync_copy(acc.at[pl.ds(sid * RPT, RPT)],
                    out_hbm.at[cid, pl.ds(sid * RPT, RPT)])


_TC_BLK = 1000


def _prep_body(f_ref, o_ref):
    x = f_ref[...].reshape(_TC_BLK, D // 32, 2, 16)
    a = jax.lax.bitcast_convert_type(
        x[:, :, 0, :].astype(jnp.bfloat16), jnp.uint16)
    b = jax.lax.bitcast_convert_type(
        x[:, :, 1, :].astype(jnp.bfloat16), jnp.uint16)
    word = (a.astype(jnp.int32)
            | (b.astype(jnp.int32) << 16)).reshape(_TC_BLK, D // 2)
    o_ref[...] = word


def _tc_prep(feat):
    return pl.pallas_call(
        _prep_body,
        grid=(N // _TC_BLK,),
        in_specs=[pl.BlockSpec((_TC_BLK, D), lambda i: (i, 0))],
        out_specs=pl.BlockSpec((_TC_BLK, D // 2), lambda i: (i, 0)),
        out_shape=jax.ShapeDtypeStruct((N, D // 2), jnp.int32),
    )(feat)


def _tc_fw_body(f_ref, w_ref, o_ref):
    o_ref[...] = jnp.dot((1.0 + EPSILON) * f_ref[...], w_ref[...],
                         preferred_element_type=jnp.float32)


def _tc_fw(feat, W):
    return pl.pallas_call(
        _tc_fw_body,
        grid=(N // _TC_BLK,),
        in_specs=[
            pl.BlockSpec((_TC_BLK, D), lambda i: (i, 0)),
            pl.BlockSpec((D, D), lambda i: (0, 0)),
        ],
        out_specs=pl.BlockSpec((_TC_BLK, D), lambda i: (i, 0)),
        out_shape=jax.ShapeDtypeStruct((N, D), jnp.float32),
    )(feat, W)


def _tc_body(fw_ref, p0_ref, p1_ref, w_ref, o_ref):
    p = p0_ref[0] + p1_ref[0]
    o_ref[...] = fw_ref[...] + jnp.dot(
        p, w_ref[...], preferred_element_type=jnp.float32)


def _tc_finish(fw, part, W):
    return pl.pallas_call(
        _tc_body,
        grid=(N // _TC_BLK,),
        in_specs=[
            pl.BlockSpec((_TC_BLK, D), lambda i: (i, 0)),
            pl.BlockSpec((1, _TC_BLK, D), lambda i: (0, i, 0)),
            pl.BlockSpec((1, _TC_BLK, D), lambda i: (1, i, 0)),
            pl.BlockSpec((D, D), lambda i: (0, 0)),
        ],
        out_specs=pl.BlockSpec((_TC_BLK, D), lambda i: (i, 0)),
        out_shape=jax.ShapeDtypeStruct((N, D), jnp.float32),
    )(fw, part, part, W)


def kernel(feat, edge_index, W):
    src = edge_index[0]
    dst = edge_index[1]
    pad = E_PAD - E
    src_p = jnp.concatenate(
        [src, jnp.arange(pad, dtype=jnp.int32) % N])
    dst_p = jnp.concatenate(
        [dst, N + (jnp.arange(pad, dtype=jnp.int32) % (ROWS_SH - N))])
    packed = (src_p | (dst_p << 16)).reshape(NW, EPW)
    part = _sc_agg(packed, _tc_prep(feat))
    return _tc_finish(_tc_fw(feat, W), part, W)


# speedup vs baseline: 1.1858x; 1.1858x over previous
import functools

import jax
import jax.numpy as jnp
from jax import lax
from jax.experimental import pallas as pl
from jax.experimental.pallas import tpu as pltpu
from jax.experimental.pallas import tpu_sc as plsc

N = 10000
D = 128
E = 320000
EPSILON = 0.1

NC = 2
NS = 16
NW = NC * NS
EPB = 64
NB = 160
RING = 4
EPW = EPB * NB
E_PAD = EPW * NW
ROWS_SH = 10240
RPT = ROWS_SH // NS

_sc_mesh = plsc.VectorSubcoreMesh(core_axis_name="c", subcore_axis_name="s")


@functools.partial(
    pl.kernel,
    mesh=_sc_mesh,
    compiler_params=pltpu.CompilerParams(
        use_tc_tiling_on_sc=False, needs_layout_passes=False),
    out_type=jax.ShapeDtypeStruct((NC, ROWS_SH, D), jnp.float32),
    scratch_types=[
        pltpu.VMEM((EPW,), jnp.int32),
        pltpu.VMEM((4, EPB), jnp.int32),
        pltpu.VMEM((4, EPB), jnp.int32),
        pltpu.VMEM((3, EPB, D // 2), jnp.int32),
        pltpu.VMEM((2, EPB, D), jnp.float32),
        pltpu.VMEM_SHARED((ROWS_SH, D), jnp.float32),
        pltpu.SemaphoreType.DMA,
        pltpu.SemaphoreType.DMA((3,)),
        pltpu.SemaphoreType.DMA((2,)),
    ],
)
def _sc_agg(pk_hbm, feat_hbm, out_hbm,
            pk_v, srcr, dstr, bufb, buff, acc, sem0, sems, sem2):
    cid = lax.axis_index("c")
    sid = lax.axis_index("s")
    wid = sid * NC + cid

    pltpu.sync_copy(pk_hbm.at[wid], pk_v)

    zero16 = jnp.zeros((16,), jnp.float32)

    def _zrow(i, carry):
        for k in range(D // 16):
            buff[0, i, pl.ds(k * 16, 16)] = zero16
        return carry

    lax.fori_loop(0, EPB, _zrow, 0)
    for b in range(RPT // EPB):
        pltpu.sync_copy(buff.at[0], acc.at[pl.ds(sid * RPT + b * EPB, EPB)])
    plsc.subcore_barrier()

    msk_hi = jnp.full((16,), -65536, dtype=jnp.int32)

    def _body(j, carry):
        pari = lax.rem(j, 4)
        parf = lax.rem(j, 2)
        parb = lax.rem(j, 3)

        @pl.when(j >= 4)
        def _wscat():
            pltpu.make_async_copy(
                buff.at[parf], acc.at[dstr.at[pari]], sem2.at[parf]).wait()

        @pl.when(jnp.logical_and(j >= 2, j < NB + 2))
        def _scat():
            parb2 = lax.rem(j + 1, 3)
            pari2 = lax.rem(j + 2, 4)
            pltpu.make_async_copy(
                feat_hbm.at[srcr.at[pari2]], bufb.at[parb2],
                sems.at[parb2]).wait()
            for r in range(EPB):
                for g in range(D // 32):
                    v = bufb[parb2, r, pl.ds(16 * g, 16)]
                    lo = plsc.bitcast(v << 16, jnp.float32)
                    hi = plsc.bitcast(v & msk_hi, jnp.float32)
                    buff[parf, r, pl.ds(32 * g, 16)] = lo
                    buff[parf, r, pl.ds(32 * g + 16, 16)] = hi
            pltpu.async_copy(
                buff.at[parf], acc.at[dstr.at[pari2]], sem2.at[parf],
                add=True)

        @pl.when(j < NB)
        def _issue():
            for k in range(EPB // 16):
                p = pk_v[pl.ds(j * EPB + k * 16, 16)]
                srcr[pari, pl.ds(k * 16, 16)] = p & 0xFFFF
                dstr[pari, pl.ds(k * 16, 16)] = p >> 16
            pltpu.async_copy(
                feat_hbm.at[srcr.at[pari]], bufb.at[parb], sems.at[parb])

        return carry

    lax.fori_loop(0, NB + 4, _body, 0)
    plsc.subcore_barrier()

    pltpu.sync_copy(acc.at[pl.ds(sid * RPT, RPT)],
                    out_hbm.at[cid, pl.ds(sid * RPT, RPT)])


_TC_BLK = 1000


def _tc_fw_body(f_ref, w_ref, o_ref):
    o_ref[...] = jnp.dot((1.0 + EPSILON) * f_ref[...], w_ref[...],
                         preferred_element_type=jnp.float32)


def _tc_fw(feat, W):
    return pl.pallas_call(
        _tc_fw_body,
        grid=(N // _TC_BLK,),
        in_specs=[
            pl.BlockSpec((_TC_BLK, D), lambda i: (i, 0)),
            pl.BlockSpec((D, D), lambda i: (0, 0)),
        ],
        out_specs=pl.BlockSpec((_TC_BLK, D), lambda i: (i, 0)),
        out_shape=jax.ShapeDtypeStruct((N, D), jnp.float32),
    )(feat, W)


def _tc_body(fw_ref, p0_ref, p1_ref, w_ref, o_ref):
    p = p0_ref[0] + p1_ref[0]
    o_ref[...] = fw_ref[...] + jnp.dot(
        p, w_ref[...], preferred_element_type=jnp.float32)


def _tc_finish(fw, part, W):
    return pl.pallas_call(
        _tc_body,
        grid=(N // _TC_BLK,),
        in_specs=[
            pl.BlockSpec((_TC_BLK, D), lambda i: (i, 0)),
            pl.BlockSpec((1, _TC_BLK, D), lambda i: (0, i, 0)),
            pl.BlockSpec((1, _TC_BLK, D), lambda i: (1, i, 0)),
            pl.BlockSpec((D, D), lambda i: (0, 0)),
        ],
        out_specs=pl.BlockSpec((_TC_BLK, D), lambda i: (i, 0)),
        out_shape=jax.ShapeDtypeStruct((N, D), jnp.float32),
    )(fw, part, part, W)


def kernel(feat, edge_index, W):
    src = edge_index[0]
    dst = edge_index[1]
    pad = E_PAD - E
    src_p = jnp.concatenate(
        [src, jnp.arange(pad, dtype=jnp.int32) % N])
    dst_p = jnp.concatenate(
        [dst, N + (jnp.arange(pad, dtype=jnp.int32) % (ROWS_SH - N))])
    packed = (src_p | (dst_p << 16)).reshape(NW, EPW)
    feat_bf = (feat.astype(jnp.bfloat16)
               .reshape(N, D // 32, 2, 16)
               .transpose(0, 1, 3, 2)
               .reshape(N, D // 2, 2))
    featq = jax.lax.bitcast_convert_type(feat_bf, jnp.int32)
    part = _sc_agg(packed, featq)
    return _tc_finish(_tc_fw(feat, W), part, W)
